# baseline (speedup 1.0000x reference)
import jax
import jax.numpy as jnp
from jax.experimental import pallas as pl
from jax.experimental.pallas import tpu as pltpu

K = 1_000_000
D = 64
B = 16_384

FV_ROWS = K * D // 128
NEW_FV_ROWS = B * D // 128
OLD_FV_ROWS = FV_ROWS - NEW_FV_ROWS

LV_ROWS = K // 64
NEW_LV_ROWS = B // 64
OLD_LV_ROWS = LV_ROWS - NEW_LV_ROWS

NCHUNK = 8
CHUNK = OLD_FV_ROWS // NCHUNK
NDMA = NCHUNK + 3


def _dma_body(feats_ref, features_ref, lnew_ref, lold_ref,
              out_f_ref, out_l_ref, sem):
    copies = [
        pltpu.make_async_copy(
            feats_ref, out_f_ref.at[pl.ds(0, NEW_FV_ROWS)], sem.at[0]),
        pltpu.make_async_copy(
            lnew_ref, out_l_ref.at[pl.ds(0, NEW_LV_ROWS)], sem.at[1]),
        pltpu.make_async_copy(
            lold_ref.at[pl.ds(NEW_LV_ROWS, OLD_LV_ROWS)],
            out_l_ref.at[pl.ds(NEW_LV_ROWS, OLD_LV_ROWS)], sem.at[2]),
    ]
    for c in range(NCHUNK):
        start = NEW_FV_ROWS + c * CHUNK
        copies.append(pltpu.make_async_copy(
            features_ref.at[pl.ds(start, CHUNK)],
            out_f_ref.at[pl.ds(start, CHUNK)], sem.at[3 + c]))
    for c in copies:
        c.start()
    for c in copies:
        c.wait()


def kernel(feats, labels, features, labels_buf):
    fv = features.reshape(FV_ROWS, 128)
    nv = feats.reshape(NEW_FV_ROWS, 128)
    lv = labels_buf.reshape(LV_ROWS, 64)
    ln = labels.reshape(NEW_LV_ROWS, 64)

    out_f, out_l = pl.pallas_call(
        _dma_body,
        in_specs=[
            pl.BlockSpec(memory_space=pl.ANY),
            pl.BlockSpec(memory_space=pl.ANY),
            pl.BlockSpec(memory_space=pl.ANY),
            pl.BlockSpec(memory_space=pl.ANY),
        ],
        out_specs=[
            pl.BlockSpec(memory_space=pl.ANY),
            pl.BlockSpec(memory_space=pl.ANY),
        ],
        out_shape=[
            jax.ShapeDtypeStruct((FV_ROWS, 128), jnp.float32),
            jax.ShapeDtypeStruct((LV_ROWS, 64), jnp.int32),
        ],
        scratch_shapes=[pltpu.SemaphoreType.DMA((NDMA,))],
    )(nv, fv, ln, lv)

    new_features = out_f.reshape(K, D)
    new_labels = out_l.reshape(K)
    new_ptr = jnp.full((1,), B % K, dtype=jnp.int32)
    return (new_features, new_labels, new_ptr)


# speedup vs baseline: 6.6485x; 6.6485x over previous
import jax
import jax.numpy as jnp
from jax.experimental import pallas as pl

K = 1_000_000
D = 64
B = 16_384

FV_ROWS = K * D // 128
NEW_FV_ROWS = B * D // 128
RF = 8_192
NB_NEW = NEW_FV_ROWS // RF
GRID = (FV_ROWS + RF - 1) // RF

LV_ROWS = K // 64
NEW_LV_ROWS = B // 64
RL = NEW_LV_ROWS // NB_NEW


def _copy_body(feats_ref, features_ref, lnew_ref, lold_ref, out_f_ref, out_l_ref):
    i = pl.program_id(0)

    @pl.when(i < NB_NEW)
    def _():
        out_f_ref[...] = feats_ref[...]
        out_l_ref[...] = lnew_ref[...]

    @pl.when(i >= NB_NEW)
    def _():
        out_f_ref[...] = features_ref[...]
        out_l_ref[...] = lold_ref[...]


def kernel(feats, labels, features, labels_buf):
    fv = features.reshape(FV_ROWS, 128)
    nv = feats.reshape(NEW_FV_ROWS, 128)
    lv = labels_buf.reshape(LV_ROWS, 64)
    ln = labels.reshape(NEW_LV_ROWS, 64)

    out_f, out_l = pl.pallas_call(
        _copy_body,
        grid=(GRID,),
        in_specs=[
            pl.BlockSpec((RF, 128), lambda i: (jnp.minimum(i, NB_NEW - 1), 0)),
            pl.BlockSpec((RF, 128), lambda i: (jnp.maximum(i, NB_NEW), 0)),
            pl.BlockSpec((RL, 64), lambda i: (jnp.minimum(i, NB_NEW - 1), 0)),
            pl.BlockSpec((RL, 64), lambda i: (jnp.maximum(i, NB_NEW), 0)),
        ],
        out_specs=[
            pl.BlockSpec((RF, 128), lambda i: (i, 0)),
            pl.BlockSpec((RL, 64), lambda i: (i, 0)),
        ],
        out_shape=[
            jax.ShapeDtypeStruct((FV_ROWS, 128), jnp.float32),
            jax.ShapeDtypeStruct((LV_ROWS, 64), jnp.int32),
        ],
    )(nv, fv, ln, lv)

    new_features = out_f.reshape(K, D)
    new_labels = out_l.reshape(K)
    new_ptr = jnp.full((1,), B % K, dtype=jnp.int32)
    return (new_features, new_labels, new_ptr)
